# baseline (speedup 1.0000x reference)
import jax
import jax.numpy as jnp
from jax import lax
from jax.experimental import pallas as pl
from jax.experimental.pallas import tpu as pltpu

N = 10000
D = 128
HID = 32
OUT = 9
BLK = 1024


def _cell_kernel(x_ref, ht_ref, ct_ref, wx_ref, wh_ref, bx_ref, bh_ref,
                 bg_ref, wc_ref, wl_ref, bl_ref, yt_ref, hnt_ref, cnt_ref):
    x = x_ref[...]
    ht = ht_ref[...]
    ct = ct_ref[...]
    wxct = wx_ref[...].reshape(4 * HID, D)
    whc = jnp.concatenate([wh_ref[0], wh_ref[1], wh_ref[2], wh_ref[3]],
                          axis=1)
    zt = (lax.dot_general(wxct, x, (((1,), (1,)), ((), ())),
                          preferred_element_type=jnp.float32)
          + lax.dot_general(whc, ht, (((0,), (0,)), ((), ())),
                            preferred_element_type=jnp.float32))
    bt = jnp.transpose(bx_ref[...] + bh_ref[...] + bg_ref[...])
    wct = jnp.transpose(wc_ref[...])
    gi = jax.nn.sigmoid(zt[0 * HID:1 * HID, :] + bt[:, 0:1] + wct[:, 0:1] * ct)
    gf = jax.nn.sigmoid(zt[1 * HID:2 * HID, :] + bt[:, 1:2] + wct[:, 1:2] * ct)
    gt = jnp.tanh(zt[2 * HID:3 * HID, :] + bt[:, 2:3])
    c_new = gf * ct + gi * gt
    go = jax.nn.sigmoid(zt[3 * HID:4 * HID, :] + bt[:, 3:4] + wct[:, 2:3] * c_new)
    h_new = go * jnp.tanh(c_new)
    yt = (jnp.dot(wl_ref[...], jnp.maximum(h_new, 0.0),
                  preferred_element_type=jnp.float32)
          + jnp.transpose(bl_ref[...]))
    yt_ref[...] = yt
    hnt_ref[...] = h_new
    cnt_ref[...] = c_new


def kernel(x, edge_index, edge_weight, h, c, Wx, bx, Wh, bh, wc, bg, Wl, bl):
    del edge_index, edge_weight
    ht = h.T
    ct = c.T
    wxt = Wx.transpose(0, 2, 1)
    wlt = Wl.T
    bl2 = bl.reshape(1, OUT)

    grid = ((N + BLK - 1) // BLK,)
    rows = lambda i: (i, 0)
    cols = lambda i: (0, i)
    fixed2 = lambda i: (0, 0)
    fixed3 = lambda i: (0, 0, 0)
    yt, hnt, cnt = pl.pallas_call(
        _cell_kernel,
        grid=grid,
        in_specs=[
            pl.BlockSpec((BLK, D), rows),
            pl.BlockSpec((HID, BLK), cols),
            pl.BlockSpec((HID, BLK), cols),
            pl.BlockSpec((4, HID, D), fixed3),
            pl.BlockSpec((4, HID, HID), fixed3),
            pl.BlockSpec((4, HID), fixed2),
            pl.BlockSpec((4, HID), fixed2),
            pl.BlockSpec((4, HID), fixed2),
            pl.BlockSpec((3, HID), fixed2),
            pl.BlockSpec((OUT, HID), fixed2),
            pl.BlockSpec((1, OUT), fixed2),
        ],
        out_specs=[
            pl.BlockSpec((OUT, BLK), cols),
            pl.BlockSpec((HID, BLK), cols),
            pl.BlockSpec((HID, BLK), cols),
        ],
        out_shape=[
            jax.ShapeDtypeStruct((OUT, N), jnp.float32),
            jax.ShapeDtypeStruct((HID, N), jnp.float32),
            jax.ShapeDtypeStruct((HID, N), jnp.float32),
        ],
        compiler_params=pltpu.CompilerParams(
            dimension_semantics=("parallel",)),
    )(x, ht, ct, wxt, Wh, bx, bh, bg, wc, wlt, bl2)
    return (yt.T, hnt.T, cnt.T)


# speedup vs baseline: 1.4839x; 1.4839x over previous
import jax
import jax.numpy as jnp
from jax import lax
from jax.experimental import pallas as pl
from jax.experimental.pallas import tpu as pltpu

N = 10000
D = 128
HID = 32
OUT = 9
BLK = 4096


def _cell_kernel(x_ref, ht_ref, ct_ref, wx_ref, wh_ref, bx_ref, bh_ref,
                 bg_ref, wc_ref, wl_ref, bl_ref, yt_ref, hnt_ref, cnt_ref):
    x = x_ref[...]
    ht = ht_ref[...]
    ct = ct_ref[...]
    wxct = wx_ref[...].reshape(4 * HID, D)
    whc = jnp.concatenate([wh_ref[0], wh_ref[1], wh_ref[2], wh_ref[3]],
                          axis=1)
    zt = (lax.dot_general(wxct, x, (((1,), (1,)), ((), ())),
                          preferred_element_type=jnp.float32)
          + lax.dot_general(whc, ht, (((0,), (0,)), ((), ())),
                            preferred_element_type=jnp.float32))
    bt = jnp.transpose(bx_ref[...] + bh_ref[...] + bg_ref[...])
    wct = jnp.transpose(wc_ref[...])
    gi = jax.nn.sigmoid(zt[0 * HID:1 * HID, :] + bt[:, 0:1] + wct[:, 0:1] * ct)
    gf = jax.nn.sigmoid(zt[1 * HID:2 * HID, :] + bt[:, 1:2] + wct[:, 1:2] * ct)
    gt = jnp.tanh(zt[2 * HID:3 * HID, :] + bt[:, 2:3])
    c_new = gf * ct + gi * gt
    go = jax.nn.sigmoid(zt[3 * HID:4 * HID, :] + bt[:, 3:4] + wct[:, 2:3] * c_new)
    h_new = go * jnp.tanh(c_new)
    yt = (jnp.dot(wl_ref[...], jnp.maximum(h_new, 0.0),
                  preferred_element_type=jnp.float32)
          + jnp.transpose(bl_ref[...]))
    yt_ref[...] = yt
    hnt_ref[...] = h_new
    cnt_ref[...] = c_new


def kernel(x, edge_index, edge_weight, h, c, Wx, bx, Wh, bh, wc, bg, Wl, bl):
    del edge_index, edge_weight
    ht = h.T
    ct = c.T
    wxt = Wx.transpose(0, 2, 1)
    wlt = Wl.T
    bl2 = bl.reshape(1, OUT)

    grid = ((N + BLK - 1) // BLK,)
    rows = lambda i: (i, 0)
    cols = lambda i: (0, i)
    fixed2 = lambda i: (0, 0)
    fixed3 = lambda i: (0, 0, 0)
    yt, hnt, cnt = pl.pallas_call(
        _cell_kernel,
        grid=grid,
        in_specs=[
            pl.BlockSpec((BLK, D), rows),
            pl.BlockSpec((HID, BLK), cols),
            pl.BlockSpec((HID, BLK), cols),
            pl.BlockSpec((4, HID, D), fixed3),
            pl.BlockSpec((4, HID, HID), fixed3),
            pl.BlockSpec((4, HID), fixed2),
            pl.BlockSpec((4, HID), fixed2),
            pl.BlockSpec((4, HID), fixed2),
            pl.BlockSpec((3, HID), fixed2),
            pl.BlockSpec((OUT, HID), fixed2),
            pl.BlockSpec((1, OUT), fixed2),
        ],
        out_specs=[
            pl.BlockSpec((OUT, BLK), cols),
            pl.BlockSpec((HID, BLK), cols),
            pl.BlockSpec((HID, BLK), cols),
        ],
        out_shape=[
            jax.ShapeDtypeStruct((OUT, N), jnp.float32),
            jax.ShapeDtypeStruct((HID, N), jnp.float32),
            jax.ShapeDtypeStruct((HID, N), jnp.float32),
        ],
        compiler_params=pltpu.CompilerParams(
            dimension_semantics=("parallel",)),
    )(x, ht, ct, wxt, Wh, bx, bh, bg, wc, wlt, bl2)
    return (yt.T, hnt.T, cnt.T)


# speedup vs baseline: 1.6557x; 1.1158x over previous
import jax
import jax.numpy as jnp
from jax import lax
from jax.experimental import pallas as pl
from jax.experimental.pallas import tpu as pltpu

N = 10000
D = 128
HID = 32
OUT = 9
BLK = 5120


def _cell_kernel(x_ref, ht_ref, ct_ref, wx_ref, wh_ref, bx_ref, bh_ref,
                 bg_ref, wc_ref, wl_ref, bl_ref, yt_ref, hnt_ref, cnt_ref):
    x = x_ref[...]
    ht = ht_ref[...]
    ct = ct_ref[...]
    wxct = wx_ref[...].reshape(4 * HID, D)
    whc = jnp.concatenate([wh_ref[0], wh_ref[1], wh_ref[2], wh_ref[3]],
                          axis=1)
    zt = (lax.dot_general(wxct, x, (((1,), (1,)), ((), ())),
                          preferred_element_type=jnp.float32)
          + lax.dot_general(whc, ht, (((0,), (0,)), ((), ())),
                            preferred_element_type=jnp.float32))
    bt = jnp.transpose(bx_ref[...] + bh_ref[...] + bg_ref[...])
    wct = jnp.transpose(wc_ref[...])
    gi = jax.nn.sigmoid(zt[0 * HID:1 * HID, :] + bt[:, 0:1] + wct[:, 0:1] * ct)
    gf = jax.nn.sigmoid(zt[1 * HID:2 * HID, :] + bt[:, 1:2] + wct[:, 1:2] * ct)
    gt = jnp.tanh(zt[2 * HID:3 * HID, :] + bt[:, 2:3])
    c_new = gf * ct + gi * gt
    go = jax.nn.sigmoid(zt[3 * HID:4 * HID, :] + bt[:, 3:4] + wct[:, 2:3] * c_new)
    h_new = go * jnp.tanh(c_new)
    yt = (jnp.dot(wl_ref[...], jnp.maximum(h_new, 0.0),
                  preferred_element_type=jnp.float32)
          + jnp.transpose(bl_ref[...]))
    yt_ref[...] = yt
    hnt_ref[...] = h_new
    cnt_ref[...] = c_new


def kernel(x, edge_index, edge_weight, h, c, Wx, bx, Wh, bh, wc, bg, Wl, bl):
    del edge_index, edge_weight
    ht = h.T
    ct = c.T
    wxt = Wx.transpose(0, 2, 1)
    wlt = Wl.T
    bl2 = bl.reshape(1, OUT)

    grid = ((N + BLK - 1) // BLK,)
    rows = lambda i: (i, 0)
    cols = lambda i: (0, i)
    fixed2 = lambda i: (0, 0)
    fixed3 = lambda i: (0, 0, 0)
    yt, hnt, cnt = pl.pallas_call(
        _cell_kernel,
        grid=grid,
        in_specs=[
            pl.BlockSpec((BLK, D), rows),
            pl.BlockSpec((HID, BLK), cols),
            pl.BlockSpec((HID, BLK), cols),
            pl.BlockSpec((4, HID, D), fixed3),
            pl.BlockSpec((4, HID, HID), fixed3),
            pl.BlockSpec((4, HID), fixed2),
            pl.BlockSpec((4, HID), fixed2),
            pl.BlockSpec((4, HID), fixed2),
            pl.BlockSpec((3, HID), fixed2),
            pl.BlockSpec((OUT, HID), fixed2),
            pl.BlockSpec((1, OUT), fixed2),
        ],
        out_specs=[
            pl.BlockSpec((OUT, BLK), cols),
            pl.BlockSpec((HID, BLK), cols),
            pl.BlockSpec((HID, BLK), cols),
        ],
        out_shape=[
            jax.ShapeDtypeStruct((OUT, N), jnp.float32),
            jax.ShapeDtypeStruct((HID, N), jnp.float32),
            jax.ShapeDtypeStruct((HID, N), jnp.float32),
        ],
        compiler_params=pltpu.CompilerParams(
            dimension_semantics=("parallel",)),
    )(x, ht, ct, wxt, Wh, bx, bh, bg, wc, wlt, bl2)
    return (yt.T, hnt.T, cnt.T)
